# Initial kernel scaffold; baseline (speedup 1.0000x reference)
import jax
import jax.numpy as jnp
from jax.experimental import pallas as pl


def kernel(x1, x2, ws1, ws2, wt1, wt2, aw, ab):
    raise NotImplementedError("write your pallas kernel here")


# speedup vs baseline: 1.0111x; 1.0111x over previous
import functools

import jax
import jax.numpy as jnp
from jax.experimental import pallas as pl
from jax.experimental.pallas import tpu as pltpu

_LANES = 128
_MAX_C_BLK = 512
_VMEM_LIMIT = 56 * 1024 * 1024


def _pool_body(x1_ref, x2_ref, o1_ref, o2_ref, *, inv_s):
    o1_ref[0] = (jnp.sum(x1_ref[0], axis=-1) * inv_s)[None, :]
    o2_ref[0] = (jnp.sum(x2_ref[0], axis=-1) * inv_s)[None, :]


def _epilogue_body(a1_ref, a2_ref, ws1_ref, ws2_ref, wt1_ref, wt2_ref,
                   awb_ref, p0_ref, p1_ref):
    hp = jax.lax.Precision.HIGHEST
    dn = (((1,), (1,)), ((), ()))
    a1 = a1_ref[...]
    a2 = a2_ref[...]
    h1 = jax.lax.dot_general(a1, ws1_ref[...], dn, precision=hp,
                             preferred_element_type=jnp.float32)
    s_out = jax.lax.dot_general(h1, ws2_ref[...], dn, precision=hp,
                                preferred_element_type=jnp.float32)
    h2 = jax.lax.dot_general(a2, wt1_ref[...], dn, precision=hp,
                             preferred_element_type=jnp.float32)
    t_out = jax.lax.dot_general(h2, wt2_ref[...], dn, precision=hp,
                                preferred_element_type=jnp.float32)
    c0 = awb_ref[0, 0] - awb_ref[1, 0]
    c1 = awb_ref[0, 1] - awb_ref[1, 1]
    cb = awb_ref[0, 2] - awb_ref[1, 2]
    d = s_out * c0 + t_out * c1 + cb
    p0 = jax.nn.sigmoid(d)
    p0_ref[...] = p0
    p1_ref[...] = 1.0 - p0


def _pick_c_blk(C):
    if C <= _MAX_C_BLK or C % _LANES != 0:
        return C
    for cand in range(_MAX_C_BLK, _LANES - 1, -_LANES):
        if C % cand == 0:
            return cand
    return C


def kernel(x1, x2, ws1, ws2, wt1, wt2, aw, ab):
    N, C, T, W, H = x1.shape
    S = T * W * H
    x1f = x1.reshape(N, C, S)
    x2f = x2.reshape(N, C, S)

    c_blk = _pick_c_blk(C)
    nc = C // c_blk
    R = N * nc

    sums1, sums2 = pl.pallas_call(
        functools.partial(_pool_body, inv_s=1.0 / S),
        out_shape=[jax.ShapeDtypeStruct((R, 1, c_blk), jnp.float32)] * 2,
        grid=(R,),
        in_specs=[
            pl.BlockSpec((1, c_blk, S), lambda r: (r // nc, r % nc, 0)),
            pl.BlockSpec((1, c_blk, S), lambda r: (r // nc, r % nc, 0)),
        ],
        out_specs=[
            pl.BlockSpec((1, 1, c_blk), lambda r: (r, 0, 0)),
            pl.BlockSpec((1, 1, c_blk), lambda r: (r, 0, 0)),
        ],
        compiler_params=pltpu.CompilerParams(
            dimension_semantics=("parallel",),
            vmem_limit_bytes=_VMEM_LIMIT,
        ),
        cost_estimate=pl.CostEstimate(
            flops=int(2 * N * C * S),
            transcendentals=0,
            bytes_accessed=int(2 * N * C * S * 4 + 2 * N * C * 4),
        ),
    )(x1f, x2f)

    a1 = sums1.reshape(N, C)
    a2 = sums2.reshape(N, C)
    awb = jnp.concatenate([aw, ab.reshape(2, 1)], axis=1)

    p0, p1 = pl.pallas_call(
        _epilogue_body,
        out_shape=[jax.ShapeDtypeStruct((N, C), jnp.float32)] * 2,
        in_specs=[
            pl.BlockSpec(a1.shape, lambda: (0, 0)),
            pl.BlockSpec(a2.shape, lambda: (0, 0)),
            pl.BlockSpec(ws1.shape, lambda: (0, 0)),
            pl.BlockSpec(ws2.shape, lambda: (0, 0)),
            pl.BlockSpec(wt1.shape, lambda: (0, 0)),
            pl.BlockSpec(wt2.shape, lambda: (0, 0)),
            pl.BlockSpec(memory_space=pltpu.SMEM),
        ],
        out_specs=[
            pl.BlockSpec((N, C), lambda: (0, 0)),
            pl.BlockSpec((N, C), lambda: (0, 0)),
        ],
        compiler_params=pltpu.CompilerParams(
            vmem_limit_bytes=_VMEM_LIMIT,
        ),
    )(a1, a2, ws1, ws2, wt1, wt2, awb)

    p = jnp.stack([p0, p1], axis=-1)
    return p.reshape(N, C, 2, 1, 1, 1)


# speedup vs baseline: 5.3825x; 5.3234x over previous
import functools

import jax
import jax.numpy as jnp
from jax.experimental import pallas as pl
from jax.experimental.pallas import tpu as pltpu

_VMEM_LIMIT = 60 * 1024 * 1024


def _pool_body(x1_ref, x2_ref, o1_ref, o2_ref, *, inv_s):
    o1_ref[0] = jnp.sum(x1_ref[0], axis=0, keepdims=True) * inv_s
    o2_ref[0] = jnp.sum(x2_ref[0], axis=0, keepdims=True) * inv_s


def _epilogue_body(a1_ref, a2_ref, ws1_ref, ws2_ref, wt1_ref, wt2_ref,
                   awb_ref, p0_ref, p1_ref):
    hp = jax.lax.Precision.HIGHEST
    dn = (((1,), (1,)), ((), ()))
    a1 = a1_ref[...]
    a2 = a2_ref[...]
    h1 = jax.lax.dot_general(a1, ws1_ref[...], dn, precision=hp,
                             preferred_element_type=jnp.float32)
    s_out = jax.lax.dot_general(h1, ws2_ref[...], dn, precision=hp,
                                preferred_element_type=jnp.float32)
    h2 = jax.lax.dot_general(a2, wt1_ref[...], dn, precision=hp,
                             preferred_element_type=jnp.float32)
    t_out = jax.lax.dot_general(h2, wt2_ref[...], dn, precision=hp,
                                preferred_element_type=jnp.float32)
    c0 = awb_ref[0, 0] - awb_ref[1, 0]
    c1 = awb_ref[0, 1] - awb_ref[1, 1]
    cb = awb_ref[0, 2] - awb_ref[1, 2]
    d = s_out * c0 + t_out * c1 + cb
    p0 = jax.nn.sigmoid(d)
    p0_ref[...] = p0
    p1_ref[...] = 1.0 - p0


def kernel(x1, x2, ws1, ws2, wt1, wt2, aw, ab):
    N, C, T, W, H = x1.shape
    S = T * W * H
    x1t = x1.transpose(0, 3, 4, 2, 1).reshape(N, S, C)
    x2t = x2.transpose(0, 3, 4, 2, 1).reshape(N, S, C)

    sums1, sums2 = pl.pallas_call(
        functools.partial(_pool_body, inv_s=1.0 / S),
        out_shape=[jax.ShapeDtypeStruct((N, 1, C), jnp.float32)] * 2,
        grid=(N,),
        in_specs=[
            pl.BlockSpec((1, S, C), lambda n: (n, 0, 0)),
            pl.BlockSpec((1, S, C), lambda n: (n, 0, 0)),
        ],
        out_specs=[
            pl.BlockSpec((1, 1, C), lambda n: (n, 0, 0)),
            pl.BlockSpec((1, 1, C), lambda n: (n, 0, 0)),
        ],
        compiler_params=pltpu.CompilerParams(
            dimension_semantics=("parallel",),
            vmem_limit_bytes=_VMEM_LIMIT,
        ),
        cost_estimate=pl.CostEstimate(
            flops=int(2 * N * C * S),
            transcendentals=0,
            bytes_accessed=int(2 * N * C * S * 4 + 2 * N * C * 4),
        ),
    )(x1t, x2t)

    a1 = sums1.reshape(N, C)
    a2 = sums2.reshape(N, C)
    awb = jnp.concatenate([aw, ab.reshape(2, 1)], axis=1)

    p0, p1 = pl.pallas_call(
        _epilogue_body,
        out_shape=[jax.ShapeDtypeStruct((N, C), jnp.float32)] * 2,
        in_specs=[
            pl.BlockSpec(a1.shape, lambda: (0, 0)),
            pl.BlockSpec(a2.shape, lambda: (0, 0)),
            pl.BlockSpec(ws1.shape, lambda: (0, 0)),
            pl.BlockSpec(ws2.shape, lambda: (0, 0)),
            pl.BlockSpec(wt1.shape, lambda: (0, 0)),
            pl.BlockSpec(wt2.shape, lambda: (0, 0)),
            pl.BlockSpec(memory_space=pltpu.SMEM),
        ],
        out_specs=[
            pl.BlockSpec((N, C), lambda: (0, 0)),
            pl.BlockSpec((N, C), lambda: (0, 0)),
        ],
        compiler_params=pltpu.CompilerParams(
            vmem_limit_bytes=_VMEM_LIMIT,
        ),
    )(a1, a2, ws1, ws2, wt1, wt2, awb)

    p = jnp.stack([p0, p1], axis=-1)
    return p.reshape(N, C, 2, 1, 1, 1)
